# Initial kernel scaffold; baseline (speedup 1.0000x reference)
import jax
import jax.numpy as jnp
from jax.experimental import pallas as pl


def kernel(x, W_feat_0, W_dir_0, W_feat_1, W_dir_1, W_feat_2, W_dir_2, W_feat_3, W_dir_3, W_feat_c, W_dir_c):
    raise NotImplementedError("write your pallas kernel here")


# speedup vs baseline: 11.8483x; 11.8483x over previous
import functools

import jax
import jax.numpy as jnp
from jax.experimental import pallas as pl
from jax.experimental.pallas import tpu as pltpu

_EPS = 1e-6
_K = 20
_NEG_SLOPE = 0.2
_NEG_INF = -3.0e38
_PREC = jax.lax.Precision.HIGHEST


def _dot(a, b, dims):
    return jax.lax.dot_general(
        a, b, (dims, ((), ())),
        preferred_element_type=jnp.float32, precision=_PREC)


def _dot_bf16(a, b, dims):
    return jax.lax.dot_general(
        a.astype(jnp.bfloat16), b.astype(jnp.bfloat16), (dims, ((), ())),
        preferred_element_type=jnp.float32)


def _topk_adjacency(pair, n, k):
    cols = jax.lax.broadcasted_iota(jnp.int32, (n, n), 1)

    def body(_, carry):
        work, acc = carry
        m = jnp.max(work, axis=1, keepdims=True)
        idx = jnp.min(jnp.where(work == m, cols, n), axis=1, keepdims=True)
        sel = cols == idx
        acc = acc + sel.astype(jnp.float32)
        work = jnp.where(sel, _NEG_INF, work)
        return work, acc

    _, acc = jax.lax.fori_loop(
        0, k, body, (pair, jnp.zeros((n, n), jnp.float32)))
    return acc


def _tree_mean_k(terms):
    vals = list(terms) + [jnp.zeros_like(terms[0])] * (32 - len(terms))
    while len(vals) > 1:
        half = len(vals) // 2
        vals = [vals[i] + vals[i + half] for i in range(half)]
    return vals[0] * jnp.float32(1.0 / _K)


def _rank_order_graph_feature(pair, x, n, k, stack_ref):
    xt3 = jnp.transpose(x[:, 0, :])
    cols = jax.lax.broadcasted_iota(jnp.int32, (n, n), 1)

    def body(t, work):
        m = jnp.max(work, axis=1, keepdims=True)
        idx = jnp.min(jnp.where(work == m, cols, n), axis=1, keepdims=True)
        sel = cols == idx
        nbr = _dot(sel.astype(jnp.float32), xt3, ((1,), (0,)))
        stack_ref[t] = nbr - xt3
        return jnp.where(sel, _NEG_INF, work)

    jax.lax.fori_loop(0, k, body, pair)
    f_diff = _tree_mean_k([stack_ref[t] for t in range(k)])
    f_cent = _tree_mean_k([xt3] * k)
    return f_diff, f_cent


def _graph_linear_body(mm, x_ref, wf_ref, wd_ref, p_ref, d_ref, st_ref,
                       stack_ref):
    c = x_ref.shape[2]
    n = x_ref.shape[3]
    x = x_ref[0]
    xf = jnp.transpose(x, (1, 0, 2)).reshape(3 * c, n)
    g = _dot_bf16(xf, xf, ((0,), (0,)))
    s = jnp.sum(xf * xf, axis=0)
    pair = 2.0 * g - s[:, None] - s[None, :]
    if c == 1:
        f_diff, f_cent = _rank_order_graph_feature(pair, x, n, _K, stack_ref)
        for v in range(3):
            fvt = jnp.concatenate(
                [f_diff[:, v:v + 1], f_cent[:, v:v + 1]], axis=1)
            p_ref[0, v] = mm(wf_ref[...], fvt, ((1,), (1,)))
            d_ref[0, v] = mm(wd_ref[...], fvt, ((1,), (1,)))
    else:
        a = _topk_adjacency(pair, n, _K)
        inv_k = 1.0 / _K
        for v in range(3):
            xv = x[v]
            sv = _dot(xv, a, ((1,), (1,))) * inv_k
            ev = sv - xv
            fv = jnp.concatenate([ev, xv], axis=0)
            pv = mm(wf_ref[...], fv, ((1,), (0,)))
            dv = mm(wd_ref[...], fv, ((1,), (0,)))
            p_ref[0, v] = pv
            d_ref[0, v] = dv
    p0, p1, p2 = p_ref[0, 0], p_ref[0, 1], p_ref[0, 2]
    norm = jnp.sqrt(p0 * p0 + p1 * p1 + p2 * p2) + _EPS
    st_ref[0] = norm


def _linear_body(x_ref, wf_ref, wd_ref, p_ref, d_ref, st_ref, stack_ref):
    x = x_ref[0]
    for v in range(3):
        xv = x[v]
        p_ref[0, v] = _dot_bf16(wf_ref[...], xv, ((1,), (0,)))
        d_ref[0, v] = _dot_bf16(wd_ref[...], xv, ((1,), (0,)))
    p0, p1, p2 = p_ref[0, 0], p_ref[0, 1], p_ref[0, 2]
    norm = jnp.sqrt(p0 * p0 + p1 * p1 + p2 * p2) + _EPS
    st_ref[0] = norm


def _leaky_outs(p_ref, d_ref, mv_ref):
    p0, p1, p2 = p_ref[0, 0], p_ref[0, 1], p_ref[0, 2]
    d0, d1, d2 = d_ref[0, 0], d_ref[0, 1], d_ref[0, 2]
    norm = jnp.sqrt(p0 * p0 + p1 * p1 + p2 * p2) + _EPS
    mean = mv_ref[:, 0:1]
    var = mv_ref[:, 1:2]
    norm_bn = (norm - mean) / jnp.sqrt(var + 1e-5)
    pp = [p0 / norm * norm_bn, p1 / norm * norm_bn, p2 / norm * norm_bn]
    dot = pp[0] * d0 + pp[1] * d1 + pp[2] * d2
    dsq = d0 * d0 + d1 * d1 + d2 * d2
    mask = (dot >= 0).astype(jnp.float32)
    coef = dot / (dsq + _EPS)
    outs = []
    for pv, dv in zip(pp, (d0, d1, d2)):
        outs.append(_NEG_SLOPE * pv
                    + (1.0 - _NEG_SLOPE) * (mask * pv
                                            + (1.0 - mask) * (pv - coef * dv)))
    return outs


def _apply_body(p_ref, d_ref, mv_ref, o_ref):
    outs = _leaky_outs(p_ref, d_ref, mv_ref)
    for v in range(3):
        o_ref[0, v] = outs[v]


def _apply_mean_body(p_ref, d_ref, mv_ref, o_ref):
    outs = _leaky_outs(p_ref, d_ref, mv_ref)
    for v in range(3):
        o_ref[0, :, v:v + 1] = jnp.mean(outs[v], axis=1, keepdims=True)


def _call_stage(body, b, c, n, o, feats, wf, wd):
    wshape = wf.shape
    return pl.pallas_call(
        body,
        grid=(b,),
        in_specs=[
            pl.BlockSpec((1, 3, c, n), lambda i: (i, 0, 0, 0)),
            pl.BlockSpec(wshape, lambda i: (0, 0)),
            pl.BlockSpec(wshape, lambda i: (0, 0)),
        ],
        out_specs=[
            pl.BlockSpec((1, 3, o, n), lambda i: (i, 0, 0, 0)),
            pl.BlockSpec((1, 3, o, n), lambda i: (i, 0, 0, 0)),
            pl.BlockSpec((1, o, n), lambda i: (i, 0, 0)),
        ],
        out_shape=[
            jax.ShapeDtypeStruct((b, 3, o, n), jnp.float32),
            jax.ShapeDtypeStruct((b, 3, o, n), jnp.float32),
            jax.ShapeDtypeStruct((b, o, n), jnp.float32),
        ],
        scratch_shapes=[pltpu.VMEM((_K, n, 3), jnp.float32)],
    )(feats, wf, wd)


def _stats_to_mv(st, b, n, o):
    mean = jnp.mean(st, axis=(0, 2))
    var = jnp.var(st, axis=(0, 2))
    return jnp.stack([mean, var] + [jnp.zeros((o,), jnp.float32)] * 6, axis=1)


def _call_apply(body, b, o, n, out_shape, out_spec, p, d, mv):
    return pl.pallas_call(
        body,
        grid=(b,),
        in_specs=[
            pl.BlockSpec((1, 3, o, n), lambda i: (i, 0, 0, 0)),
            pl.BlockSpec((1, 3, o, n), lambda i: (i, 0, 0, 0)),
            pl.BlockSpec((o, 8), lambda i: (0, 0)),
        ],
        out_specs=[out_spec],
        out_shape=[out_shape],
    )(p, d, mv)[0]


def kernel(x, W_feat_0, W_dir_0, W_feat_1, W_dir_1, W_feat_2, W_dir_2,
           W_feat_3, W_dir_3, W_feat_c, W_dir_c):
    b, n, _ = x.shape
    o = W_feat_0.shape[0]
    feats = jnp.transpose(x, (0, 2, 1))[:, :, None, :]
    layer_ws = [(W_feat_0, W_dir_0), (W_feat_1, W_dir_1),
                (W_feat_2, W_dir_2), (W_feat_3, W_dir_3)]
    outs = []
    for li, (wf, wd) in enumerate(layer_ws):
        c = wf.shape[1] // 2
        body = functools.partial(_graph_linear_body, _dot_bf16)
        p, d, st = _call_stage(body, b, c, n, o, feats, wf, wd)
        mv = _stats_to_mv(st, b, n, o)
        feats = _call_apply(
            _apply_body, b, o, n,
            jax.ShapeDtypeStruct((b, 3, o, n), jnp.float32),
            pl.BlockSpec((1, 3, o, n), lambda i: (i, 0, 0, 0)),
            p, d, mv)
        outs.append(feats)
    xcat = jnp.concatenate(outs, axis=2)
    p, d, st = _call_stage(_linear_body, b, 4 * o, n, o, xcat,
                           W_feat_c, W_dir_c)
    mv = _stats_to_mv(st, b, n, o)
    out8 = _call_apply(
        _apply_mean_body, b, o, n,
        jax.ShapeDtypeStruct((b, o, 8), jnp.float32),
        pl.BlockSpec((1, o, 8), lambda i: (i, 0, 0)),
        p, d, mv)
    return out8[:, :, :3]
